# Initial kernel scaffold; baseline (speedup 1.0000x reference)
import jax
import jax.numpy as jnp
from jax.experimental import pallas as pl


def kernel(x, A_param, W1, b1, W2, b2):
    raise NotImplementedError("write your pallas kernel here")


# speedup vs baseline: 1.7251x; 1.7251x over previous
import jax
import jax.numpy as jnp
from jax.experimental import pallas as pl
from jax.experimental.pallas import tpu as pltpu

N = 4096
F = 512
H = 16
C = 16
B = 512
I = N // B
D = I // 2 + 1


def _pair(i, d):
    j = jax.lax.rem(i + d, I)
    return jnp.minimum(i, j), jnp.maximum(i, j)


def _block_T(ap_ref, d):
    u = jnp.clip(ap_ref[...], 0.0, 1.0)
    r = jax.lax.broadcasted_iota(jnp.int32, (B, B), 0)
    c = jax.lax.broadcasted_iota(jnp.int32, (B, B), 1)
    return jnp.where((d > 0) | (r < c), u, 0.0)


def _scale(d):
    return jnp.where(d == I // 2, 0.5, 1.0)


def _deg_body(ap_ref, dis_ref, acc):
    i, d = pl.program_id(0), pl.program_id(1)
    bi, bj = _pair(i, d)

    @pl.when((i == 0) & (d == 0))
    def _init():
        acc[...] = jnp.zeros_like(acc)

    T = _block_T(ap_ref, d) * _scale(d)
    ones = jnp.ones((1, B), jnp.float32)
    cs = jax.lax.dot_general(ones, T, (((1,), (0,)), ((), ())),
                             preferred_element_type=jnp.float32)
    rs = jax.lax.dot_general(ones, T, (((1,), (1,)), ((), ())),
                             preferred_element_type=jnp.float32)
    acc[0, pl.ds(bj * B, B)] += cs[0, :]
    acc[0, pl.ds(bi * B, B)] += rs[0, :]

    @pl.when((i == I - 1) & (d == D - 1))
    def _fini():
        dis_ref[...] = jax.lax.rsqrt(acc[...] + 1.0)


def _layer_body(first_mm, last_mm):
    def body(dis_ref, dense_refs, ap_ref, out_ref, u_s, acc_s):
        i, d = pl.program_id(0), pl.program_id(1)
        bi, bj = _pair(i, d)

        @pl.when((i == 0) & (d == 0))
        def _init():
            u_s[...] = dis_ref[...] * first_mm(dense_refs)
            acc_s[...] = jnp.zeros_like(acc_s)

        T = _block_T(ap_ref, d)
        s = _scale(d)
        vi = u_s[pl.ds(bi * B, B), :]
        vj = u_s[pl.ds(bj * B, B), :]
        acc_s[pl.ds(bi * B, B), :] += s * jnp.dot(
            T, vj, preferred_element_type=jnp.float32)
        acc_s[pl.ds(bj * B, B), :] += s * jax.lax.dot_general(
            T, vi, (((0,), (0,)), ((), ())),
            preferred_element_type=jnp.float32)

        @pl.when((i == I - 1) & (d == D - 1))
        def _fini():
            y = dis_ref[...] * (acc_s[...] + u_s[...])
            out_ref[...] = last_mm(dense_refs, y)

    return body


def _full(shape):
    return pl.BlockSpec(shape, lambda i, d: (0,) * len(shape))


def _ap_spec():
    return pl.BlockSpec((B, B), lambda i, d: _pair(i, d))


def kernel(x, A_param, W1, b1, W2, b2):
    assert x.shape == (N, F) and A_param.shape == (N, N)
    b1r = b1.reshape(1, H)
    b2r = b2.reshape(1, C)

    grid = (I, D)

    dis_row = pl.pallas_call(
        _deg_body,
        grid=grid,
        in_specs=[_ap_spec()],
        out_specs=_full((1, N)),
        out_shape=jax.ShapeDtypeStruct((1, N), jnp.float32),
        scratch_shapes=[
            pltpu.VMEM((1, N), jnp.float32),
        ],
    )(A_param)
    dis = dis_row.reshape(N, 1)

    def l1_first(refs):
        x_ref, w1_ref, b1_ref, w2_ref = refs
        return jnp.dot(x_ref[...], w1_ref[...],
                       preferred_element_type=jnp.float32)

    def l1_last(refs, y):
        x_ref, w1_ref, b1_ref, w2_ref = refs
        h = jax.nn.relu(y + b1_ref[...])
        return jnp.dot(h, w2_ref[...], preferred_element_type=jnp.float32)

    def body1(dis_ref, x_ref, w1_ref, b1_ref, w2_ref, ap_ref, out_ref,
              u_s, acc_s):
        _layer_body(l1_first, l1_last)(
            dis_ref, (x_ref, w1_ref, b1_ref, w2_ref), ap_ref, out_ref,
            u_s, acc_s)

    u2 = pl.pallas_call(
        body1,
        grid=grid,
        in_specs=[_full((N, 1)), _full((N, F)), _full((F, H)),
                  _full((1, H)), _full((H, C)), _ap_spec()],
        out_specs=_full((N, C)),
        out_shape=jax.ShapeDtypeStruct((N, C), jnp.float32),
        scratch_shapes=[
            pltpu.VMEM((N, H), jnp.float32),
            pltpu.VMEM((N, H), jnp.float32),
        ],
    )(dis, x, W1, b1r, W2, A_param)

    def l2_first(refs):
        (v2_ref, b2_ref) = refs
        return v2_ref[...]

    def l2_last(refs, y):
        (v2_ref, b2_ref) = refs
        return y + b2_ref[...]

    def body2(dis_ref, v2_ref, b2_ref, ap_ref, out_ref, u_s, acc_s):
        _layer_body(l2_first, l2_last)(
            dis_ref, (v2_ref, b2_ref), ap_ref, out_ref, u_s, acc_s)

    out = pl.pallas_call(
        body2,
        grid=grid,
        in_specs=[_full((N, 1)), _full((N, C)), _full((1, C)), _ap_spec()],
        out_specs=_full((N, C)),
        out_shape=jax.ShapeDtypeStruct((N, C), jnp.float32),
        scratch_shapes=[
            pltpu.VMEM((N, C), jnp.float32),
            pltpu.VMEM((N, C), jnp.float32),
        ],
    )(dis, u2, b2r, A_param)

    return out


# speedup vs baseline: 1.7456x; 1.0119x over previous
import jax
import jax.numpy as jnp
from jax.experimental import pallas as pl
from jax.experimental.pallas import tpu as pltpu

N = 4096
F = 512
H = 16
C = 16
B = 512
I = N // B
D = I // 2 + 1


def _pair(i, d):
    j = jax.lax.rem(i + d, I)
    return jnp.minimum(i, j), jnp.maximum(i, j)


def _block_T(ap_ref, d):
    u = jnp.clip(ap_ref[...], 0.0, 1.0)
    r = jax.lax.broadcasted_iota(jnp.int32, (B, B), 0)
    c = jax.lax.broadcasted_iota(jnp.int32, (B, B), 1)
    return jnp.where((d > 0) | (r < c), u, 0.0)


def _scale(d):
    return jnp.where(d == I // 2, 0.5, 1.0)


def _deg_body(ap_ref, dis_ref, acc):
    i, d = pl.program_id(0), pl.program_id(1)
    bi, bj = _pair(i, d)

    @pl.when((i == 0) & (d == 0))
    def _init():
        acc[...] = jnp.zeros_like(acc)

    T = _block_T(ap_ref, d) * _scale(d)
    ones = jnp.ones((1, B), jnp.bfloat16)
    cs = jnp.sum(T, axis=0)
    rs = jax.lax.dot_general(ones, T.astype(jnp.bfloat16),
                             (((1,), (1,)), ((), ())),
                             preferred_element_type=jnp.float32)
    acc[0, pl.ds(bj * B, B)] += cs
    acc[0, pl.ds(bi * B, B)] += rs[0, :]

    @pl.when((i == I - 1) & (d == D - 1))
    def _fini():
        dis_ref[...] = jax.lax.rsqrt(acc[...] + 1.0)


def _layer_body(first_mm, last_mm):
    def body(dis_ref, dense_refs, ap_ref, out_ref, u_s, acc_s):
        i, d = pl.program_id(0), pl.program_id(1)
        bi, bj = _pair(i, d)

        @pl.when((i == 0) & (d == 0))
        def _init():
            u_s[...] = dis_ref[...] * first_mm(dense_refs)
            acc_s[...] = jnp.zeros_like(acc_s)

        T = _block_T(ap_ref, d).astype(jnp.bfloat16)
        s = _scale(d)
        vi = u_s[pl.ds(bi * B, B), :].astype(jnp.bfloat16)
        vj = u_s[pl.ds(bj * B, B), :].astype(jnp.bfloat16)
        acc_s[pl.ds(bi * B, B), :] += s * jnp.dot(
            T, vj, preferred_element_type=jnp.float32)
        acc_s[pl.ds(bj * B, B), :] += s * jax.lax.dot_general(
            T, vi, (((0,), (0,)), ((), ())),
            preferred_element_type=jnp.float32)

        @pl.when((i == I - 1) & (d == D - 1))
        def _fini():
            y = dis_ref[...] * (acc_s[...] + u_s[...])
            out_ref[...] = last_mm(dense_refs, y)

    return body


def _full(shape):
    return pl.BlockSpec(shape, lambda i, d: (0,) * len(shape))


def _ap_spec():
    return pl.BlockSpec((B, B), lambda i, d: _pair(i, d))


def kernel(x, A_param, W1, b1, W2, b2):
    assert x.shape == (N, F) and A_param.shape == (N, N)
    b1r = b1.reshape(1, H)
    b2r = b2.reshape(1, C)

    grid = (I, D)

    dis_row = pl.pallas_call(
        _deg_body,
        grid=grid,
        in_specs=[_ap_spec()],
        out_specs=_full((1, N)),
        out_shape=jax.ShapeDtypeStruct((1, N), jnp.float32),
        scratch_shapes=[
            pltpu.VMEM((1, N), jnp.float32),
        ],
    )(A_param)
    dis = dis_row.reshape(N, 1)

    def l1_first(refs):
        x_ref, w1_ref, b1_ref, w2_ref = refs
        return jnp.dot(x_ref[...], w1_ref[...],
                       preferred_element_type=jnp.float32)

    def l1_last(refs, y):
        x_ref, w1_ref, b1_ref, w2_ref = refs
        h = jax.nn.relu(y + b1_ref[...])
        return jnp.dot(h, w2_ref[...], preferred_element_type=jnp.float32)

    def body1(dis_ref, x_ref, w1_ref, b1_ref, w2_ref, ap_ref, out_ref,
              u_s, acc_s):
        _layer_body(l1_first, l1_last)(
            dis_ref, (x_ref, w1_ref, b1_ref, w2_ref), ap_ref, out_ref,
            u_s, acc_s)

    u2 = pl.pallas_call(
        body1,
        grid=grid,
        in_specs=[_full((N, 1)), _full((N, F)), _full((F, H)),
                  _full((1, H)), _full((H, C)), _ap_spec()],
        out_specs=_full((N, C)),
        out_shape=jax.ShapeDtypeStruct((N, C), jnp.float32),
        scratch_shapes=[
            pltpu.VMEM((N, H), jnp.float32),
            pltpu.VMEM((N, H), jnp.float32),
        ],
    )(dis, x, W1, b1r, W2, A_param)

    def l2_first(refs):
        (v2_ref, b2_ref) = refs
        return v2_ref[...]

    def l2_last(refs, y):
        (v2_ref, b2_ref) = refs
        return y + b2_ref[...]

    def body2(dis_ref, v2_ref, b2_ref, ap_ref, out_ref, u_s, acc_s):
        _layer_body(l2_first, l2_last)(
            dis_ref, (v2_ref, b2_ref), ap_ref, out_ref, u_s, acc_s)

    out = pl.pallas_call(
        body2,
        grid=grid,
        in_specs=[_full((N, 1)), _full((N, C)), _full((1, C)), _ap_spec()],
        out_specs=_full((N, C)),
        out_shape=jax.ShapeDtypeStruct((N, C), jnp.float32),
        scratch_shapes=[
            pltpu.VMEM((N, C), jnp.float32),
            pltpu.VMEM((N, C), jnp.float32),
        ],
    )(dis, u2, b2r, A_param)

    return out


# speedup vs baseline: 1.8416x; 1.0550x over previous
import jax
import jax.numpy as jnp
from jax.experimental import pallas as pl
from jax.experimental.pallas import tpu as pltpu

N = 4096
F = 512
H = 16
C = 16
B = 512
I = N // B
D = I // 2 + 1


def _pair(i, d):
    j = jax.lax.rem(i + d, I)
    return jnp.minimum(i, j), jnp.maximum(i, j)


def _valid(i, d):
    return jnp.logical_not((d == D - 1) & (i >= I // 2))


def _block_T(ap_ref, d):
    u = jnp.clip(ap_ref[...], 0.0, 1.0)
    r = jax.lax.broadcasted_iota(jnp.int32, (B, B), 0)
    c = jax.lax.broadcasted_iota(jnp.int32, (B, B), 1)
    return jnp.where((d > 0) | (r < c), u, 0.0)


def _deg_body(ap_ref, dis_ref, acc):
    i, d = pl.program_id(0), pl.program_id(1)
    bi, bj = _pair(i, d)

    @pl.when((i == 0) & (d == 0))
    def _init():
        acc[...] = jnp.zeros_like(acc)

    @pl.when(_valid(i, d))
    def _compute():
        T = _block_T(ap_ref, d)
        ones = jnp.ones((1, B), jnp.bfloat16)
        cs = jnp.sum(T, axis=0)
        rs = jax.lax.dot_general(ones, T.astype(jnp.bfloat16),
                                 (((1,), (1,)), ((), ())),
                                 preferred_element_type=jnp.float32)
        acc[0, pl.ds(bj * B, B)] += cs
        acc[0, pl.ds(bi * B, B)] += rs[0, :]

    @pl.when((i == I - 1) & (d == D - 1))
    def _fini():
        dis_ref[...] = jax.lax.rsqrt(acc[...] + 1.0)


def _layer_body(first_mm, last_mm):
    def body(dis_ref, dense_refs, ap_ref, out_ref, u_s, acc_s, accT_s):
        i, d = pl.program_id(0), pl.program_id(1)
        bi, bj = _pair(i, d)

        @pl.when((i == 0) & (d == 0))
        def _init():
            u_s[...] = dis_ref[...] * first_mm(dense_refs)
            acc_s[...] = jnp.zeros_like(acc_s)
            accT_s[...] = jnp.zeros_like(accT_s)

        @pl.when(_valid(i, d))
        def _compute():
            T = _block_T(ap_ref, d).astype(jnp.bfloat16)
            vi = u_s[pl.ds(bi * B, B), :].astype(jnp.bfloat16)
            vj = u_s[pl.ds(bj * B, B), :].astype(jnp.bfloat16)
            acc_s[pl.ds(bi * B, B), :] += jnp.dot(
                T, vj, preferred_element_type=jnp.float32)
            accT_s[:, pl.ds(bj * B, B)] += jax.lax.dot_general(
                vi, T, (((0,), (0,)), ((), ())),
                preferred_element_type=jnp.float32)

        @pl.when((i == I - 1) & (d == D - 1))
        def _fini():
            tot = acc_s[...] + accT_s[...].T + u_s[...]
            out_ref[...] = last_mm(dense_refs, dis_ref[...] * tot)

    return body


def _full(shape):
    return pl.BlockSpec(shape, lambda i, d: (0,) * len(shape))


def _ap_spec():
    return pl.BlockSpec((B, B), lambda i, d: _pair(i, d))


def kernel(x, A_param, W1, b1, W2, b2):
    assert x.shape == (N, F) and A_param.shape == (N, N)
    b1r = b1.reshape(1, H)
    b2r = b2.reshape(1, C)

    grid = (I, D)

    dis_row = pl.pallas_call(
        _deg_body,
        grid=grid,
        in_specs=[_ap_spec()],
        out_specs=_full((1, N)),
        out_shape=jax.ShapeDtypeStruct((1, N), jnp.float32),
        scratch_shapes=[
            pltpu.VMEM((1, N), jnp.float32),
        ],
    )(A_param)
    dis = dis_row.reshape(N, 1)

    def l1_first(refs):
        x_ref, w1_ref, b1_ref, w2_ref = refs
        return jnp.dot(x_ref[...], w1_ref[...],
                       preferred_element_type=jnp.float32)

    def l1_last(refs, y):
        x_ref, w1_ref, b1_ref, w2_ref = refs
        h = jax.nn.relu(y + b1_ref[...])
        return jnp.dot(h, w2_ref[...], preferred_element_type=jnp.float32)

    def body1(dis_ref, x_ref, w1_ref, b1_ref, w2_ref, ap_ref, out_ref,
              u_s, acc_s, accT_s):
        _layer_body(l1_first, l1_last)(
            dis_ref, (x_ref, w1_ref, b1_ref, w2_ref), ap_ref, out_ref,
            u_s, acc_s, accT_s)

    v2 = pl.pallas_call(
        body1,
        grid=grid,
        in_specs=[_full((N, 1)), _full((N, F)), _full((F, H)),
                  _full((1, H)), _full((H, C)), _ap_spec()],
        out_specs=_full((N, C)),
        out_shape=jax.ShapeDtypeStruct((N, C), jnp.float32),
        scratch_shapes=[
            pltpu.VMEM((N, H), jnp.float32),
            pltpu.VMEM((N, H), jnp.float32),
            pltpu.VMEM((H, N), jnp.float32),
        ],
    )(dis, x, W1, b1r, W2, A_param)

    def l2_first(refs):
        (v2_ref, b2_ref) = refs
        return v2_ref[...]

    def l2_last(refs, y):
        (v2_ref, b2_ref) = refs
        return y + b2_ref[...]

    def body2(dis_ref, v2_ref, b2_ref, ap_ref, out_ref, u_s, acc_s, accT_s):
        _layer_body(l2_first, l2_last)(
            dis_ref, (v2_ref, b2_ref), ap_ref, out_ref, u_s, acc_s, accT_s)

    out = pl.pallas_call(
        body2,
        grid=grid,
        in_specs=[_full((N, 1)), _full((N, C)), _full((1, C)), _ap_spec()],
        out_specs=_full((N, C)),
        out_shape=jax.ShapeDtypeStruct((N, C), jnp.float32),
        scratch_shapes=[
            pltpu.VMEM((N, C), jnp.float32),
            pltpu.VMEM((N, C), jnp.float32),
            pltpu.VMEM((C, N), jnp.float32),
        ],
    )(dis, v2, b2r, A_param)

    return out


# speedup vs baseline: 1.8475x; 1.0032x over previous
import jax
import jax.numpy as jnp
from jax.experimental import pallas as pl
from jax.experimental.pallas import tpu as pltpu

N = 4096
F = 512
H = 16
C = 16
B = 512
I = N // B
D = I // 2 + 1


def _pair(i, d):
    j = jax.lax.rem(i + d, I)
    return jnp.minimum(i, j), jnp.maximum(i, j)


def _valid(i, d):
    return jnp.logical_not((d == D - 1) & (i >= I // 2))


def _clip_block(ap_ref, masked):
    u = jnp.clip(ap_ref[...], 0.0, 1.0)
    if masked:
        r = jax.lax.broadcasted_iota(jnp.int32, (B, B), 0)
        c = jax.lax.broadcasted_iota(jnp.int32, (B, B), 1)
        u = jnp.where(r < c, u, 0.0)
    return u


def _deg_body(ap_ref, dis_ref, acc, accc):
    i, d = pl.program_id(0), pl.program_id(1)
    bi, bj = _pair(i, d)

    @pl.when((i == 0) & (d == 0))
    def _init():
        acc[...] = jnp.zeros_like(acc)
        accc[...] = jnp.zeros_like(accc)

    def contrib(masked):
        T = _clip_block(ap_ref, masked)
        acc[0, pl.ds(bj * B, B)] += jnp.sum(T, axis=0)
        accc[pl.ds(bi * B, B), :] += jnp.sum(T, axis=1, keepdims=True)

    @pl.when(_valid(i, d) & (d == 0))
    def _diag():
        contrib(True)

    @pl.when(_valid(i, d) & (d > 0))
    def _off():
        contrib(False)

    @pl.when((i == I - 1) & (d == D - 1))
    def _fini():
        dis_ref[...] = jax.lax.rsqrt(acc[...] + accc[...].T + 1.0)


def _layer_body(first_mm, last_mm):
    def body(dis_ref, dense_refs, ap_ref, out_ref, u_s, uT_s, acc_s, accT_s):
        i, d = pl.program_id(0), pl.program_id(1)
        bi, bj = _pair(i, d)

        @pl.when((i == 0) & (d == 0))
        def _init():
            u = dis_ref[...] * first_mm(dense_refs)
            u_s[...] = u
            uT_s[...] = u.astype(jnp.bfloat16).T
            acc_s[...] = jnp.zeros_like(acc_s)
            accT_s[...] = jnp.zeros_like(accT_s)

        def contrib(masked):
            T = _clip_block(ap_ref, masked).astype(jnp.bfloat16)
            vj = u_s[pl.ds(bj * B, B), :].astype(jnp.bfloat16)
            viT = uT_s[:, pl.ds(bi * B, B)]
            acc_s[pl.ds(bi * B, B), :] += jnp.dot(
                T, vj, preferred_element_type=jnp.float32)
            accT_s[:, pl.ds(bj * B, B)] += jnp.dot(
                viT, T, preferred_element_type=jnp.float32)

        @pl.when(_valid(i, d) & (d == 0))
        def _diag():
            contrib(True)

        @pl.when(_valid(i, d) & (d > 0))
        def _off():
            contrib(False)

        @pl.when((i == I - 1) & (d == D - 1))
        def _fini():
            tot = acc_s[...] + accT_s[...].T + u_s[...]
            out_ref[...] = last_mm(dense_refs, dis_ref[...] * tot)

    return body


def _full(shape):
    return pl.BlockSpec(shape, lambda i, d: (0,) * len(shape))


def _ap_spec():
    return pl.BlockSpec((B, B), lambda i, d: _pair(i, d))


def kernel(x, A_param, W1, b1, W2, b2):
    assert x.shape == (N, F) and A_param.shape == (N, N)
    b1r = b1.reshape(1, H)
    b2r = b2.reshape(1, C)

    grid = (I, D)

    dis_row = pl.pallas_call(
        _deg_body,
        grid=grid,
        in_specs=[_ap_spec()],
        out_specs=_full((1, N)),
        out_shape=jax.ShapeDtypeStruct((1, N), jnp.float32),
        scratch_shapes=[
            pltpu.VMEM((1, N), jnp.float32),
            pltpu.VMEM((N, 1), jnp.float32),
        ],
    )(A_param)
    dis = dis_row.reshape(N, 1)

    def l1_first(refs):
        x_ref, w1_ref, b1_ref, w2_ref = refs
        return jnp.dot(x_ref[...], w1_ref[...],
                       preferred_element_type=jnp.float32)

    def l1_last(refs, y):
        x_ref, w1_ref, b1_ref, w2_ref = refs
        h = jax.nn.relu(y + b1_ref[...])
        return jnp.dot(h, w2_ref[...], preferred_element_type=jnp.float32)

    def body1(dis_ref, x_ref, w1_ref, b1_ref, w2_ref, ap_ref, out_ref,
              u_s, uT_s, acc_s, accT_s):
        _layer_body(l1_first, l1_last)(
            dis_ref, (x_ref, w1_ref, b1_ref, w2_ref), ap_ref, out_ref,
            u_s, uT_s, acc_s, accT_s)

    v2 = pl.pallas_call(
        body1,
        grid=grid,
        in_specs=[_full((N, 1)), _full((N, F)), _full((F, H)),
                  _full((1, H)), _full((H, C)), _ap_spec()],
        out_specs=_full((N, C)),
        out_shape=jax.ShapeDtypeStruct((N, C), jnp.float32),
        scratch_shapes=[
            pltpu.VMEM((N, H), jnp.float32),
            pltpu.VMEM((H, N), jnp.bfloat16),
            pltpu.VMEM((N, H), jnp.float32),
            pltpu.VMEM((H, N), jnp.float32),
        ],
    )(dis, x, W1, b1r, W2, A_param)

    def l2_first(refs):
        (v2_ref, b2_ref) = refs
        return v2_ref[...]

    def l2_last(refs, y):
        (v2_ref, b2_ref) = refs
        return y + b2_ref[...]

    def body2(dis_ref, v2_ref, b2_ref, ap_ref, out_ref,
              u_s, uT_s, acc_s, accT_s):
        _layer_body(l2_first, l2_last)(
            dis_ref, (v2_ref, b2_ref), ap_ref, out_ref,
            u_s, uT_s, acc_s, accT_s)

    out = pl.pallas_call(
        body2,
        grid=grid,
        in_specs=[_full((N, 1)), _full((N, C)), _full((1, C)), _ap_spec()],
        out_specs=_full((N, C)),
        out_shape=jax.ShapeDtypeStruct((N, C), jnp.float32),
        scratch_shapes=[
            pltpu.VMEM((N, C), jnp.float32),
            pltpu.VMEM((C, N), jnp.bfloat16),
            pltpu.VMEM((N, C), jnp.float32),
            pltpu.VMEM((C, N), jnp.float32),
        ],
    )(dis, v2, b2r, A_param)

    return out


# speedup vs baseline: 1.9806x; 1.0720x over previous
import jax
import jax.numpy as jnp
from jax.experimental import pallas as pl
from jax.experimental.pallas import tpu as pltpu

N = 4096
F = 512
H = 16
C = 16
B = 512
I = N // B
P = I * (I + 1) // 2


def _pair(k):
    bi = jnp.int32(0)
    for t in range(1, I):
        bi = bi + (k >= t * I - t * (t - 1) // 2).astype(jnp.int32)
    bj = k - (bi * I - bi * (bi - 1) // 2) + bi
    return bi, bj


def _clip_block(ap_ref, masked):
    u = jnp.clip(ap_ref[...], 0.0, 1.0)
    if masked:
        r = jax.lax.broadcasted_iota(jnp.int32, (B, B), 0)
        c = jax.lax.broadcasted_iota(jnp.int32, (B, B), 1)
        u = jnp.where(r < c, u, 0.0)
    return u


def _deg_body(ap_ref, dis_ref, acc, accc):
    k = pl.program_id(0)
    bi, bj = _pair(k)

    @pl.when(k == 0)
    def _init():
        acc[...] = jnp.zeros_like(acc)
        accc[...] = jnp.zeros_like(accc)

    def contrib(masked):
        T = _clip_block(ap_ref, masked)
        acc[0, pl.ds(bj * B, B)] += jnp.sum(T, axis=0)
        accc[pl.ds(bi * B, B), :] += jnp.sum(T, axis=1, keepdims=True)

    @pl.when(bi == bj)
    def _diag():
        contrib(True)

    @pl.when(bi != bj)
    def _off():
        contrib(False)

    @pl.when(k == P - 1)
    def _fini():
        dis_ref[...] = jax.lax.rsqrt(acc[...] + accc[...].T + 1.0)


def _layer_body(first_mm, last_mm):
    def body(dis_ref, dense_refs, ap_ref, out_ref, u_s, uT_s, acc_s, accT_s):
        k = pl.program_id(0)
        bi, bj = _pair(k)

        @pl.when(k == 0)
        def _init():
            u = dis_ref[...] * first_mm(dense_refs)
            u_s[...] = u
            uT_s[...] = u.astype(jnp.bfloat16).T
            acc_s[...] = jnp.zeros_like(acc_s)
            accT_s[...] = jnp.zeros_like(accT_s)

        def contrib(masked):
            T = _clip_block(ap_ref, masked).astype(jnp.bfloat16)
            vj = u_s[pl.ds(bj * B, B), :].astype(jnp.bfloat16)
            viT = uT_s[:, pl.ds(bi * B, B)]
            acc_s[pl.ds(bi * B, B), :] += jnp.dot(
                T, vj, preferred_element_type=jnp.float32)
            accT_s[:, pl.ds(bj * B, B)] += jnp.dot(
                viT, T, preferred_element_type=jnp.float32)

        @pl.when(bi == bj)
        def _diag():
            contrib(True)

        @pl.when(bi != bj)
        def _off():
            contrib(False)

        @pl.when(k == P - 1)
        def _fini():
            tot = acc_s[...] + accT_s[...].T + u_s[...]
            out_ref[...] = last_mm(dense_refs, dis_ref[...] * tot)

    return body


def _full(shape):
    return pl.BlockSpec(shape, lambda k: (0,) * len(shape))


def _ap_spec():
    return pl.BlockSpec((B, B), _pair)


def kernel(x, A_param, W1, b1, W2, b2):
    assert x.shape == (N, F) and A_param.shape == (N, N)
    b1r = b1.reshape(1, H)
    b2r = b2.reshape(1, C)

    grid = (P,)

    dis_row = pl.pallas_call(
        _deg_body,
        grid=grid,
        in_specs=[_ap_spec()],
        out_specs=_full((1, N)),
        out_shape=jax.ShapeDtypeStruct((1, N), jnp.float32),
        scratch_shapes=[
            pltpu.VMEM((1, N), jnp.float32),
            pltpu.VMEM((N, 1), jnp.float32),
        ],
    )(A_param)
    dis = dis_row.reshape(N, 1)

    def l1_first(refs):
        x_ref, w1_ref, b1_ref, w2_ref = refs
        return jnp.dot(x_ref[...], w1_ref[...],
                       preferred_element_type=jnp.float32)

    def l1_last(refs, y):
        x_ref, w1_ref, b1_ref, w2_ref = refs
        h = jax.nn.relu(y + b1_ref[...])
        return jnp.dot(h, w2_ref[...], preferred_element_type=jnp.float32)

    def body1(dis_ref, x_ref, w1_ref, b1_ref, w2_ref, ap_ref, out_ref,
              u_s, uT_s, acc_s, accT_s):
        _layer_body(l1_first, l1_last)(
            dis_ref, (x_ref, w1_ref, b1_ref, w2_ref), ap_ref, out_ref,
            u_s, uT_s, acc_s, accT_s)

    v2 = pl.pallas_call(
        body1,
        grid=grid,
        in_specs=[_full((N, 1)), _full((N, F)), _full((F, H)),
                  _full((1, H)), _full((H, C)), _ap_spec()],
        out_specs=_full((N, C)),
        out_shape=jax.ShapeDtypeStruct((N, C), jnp.float32),
        scratch_shapes=[
            pltpu.VMEM((N, H), jnp.float32),
            pltpu.VMEM((H, N), jnp.bfloat16),
            pltpu.VMEM((N, H), jnp.float32),
            pltpu.VMEM((H, N), jnp.float32),
        ],
    )(dis, x, W1, b1r, W2, A_param)

    def l2_first(refs):
        (v2_ref, b2_ref) = refs
        return v2_ref[...]

    def l2_last(refs, y):
        (v2_ref, b2_ref) = refs
        return y + b2_ref[...]

    def body2(dis_ref, v2_ref, b2_ref, ap_ref, out_ref,
              u_s, uT_s, acc_s, accT_s):
        _layer_body(l2_first, l2_last)(
            dis_ref, (v2_ref, b2_ref), ap_ref, out_ref,
            u_s, uT_s, acc_s, accT_s)

    out = pl.pallas_call(
        body2,
        grid=grid,
        in_specs=[_full((N, 1)), _full((N, C)), _full((1, C)), _ap_spec()],
        out_specs=_full((N, C)),
        out_shape=jax.ShapeDtypeStruct((N, C), jnp.float32),
        scratch_shapes=[
            pltpu.VMEM((N, C), jnp.float32),
            pltpu.VMEM((C, N), jnp.bfloat16),
            pltpu.VMEM((N, C), jnp.float32),
            pltpu.VMEM((C, N), jnp.float32),
        ],
    )(dis, v2, b2r, A_param)

    return out


# speedup vs baseline: 3.1116x; 1.5710x over previous
import jax
import jax.numpy as jnp
from jax.experimental import pallas as pl
from jax.experimental.pallas import tpu as pltpu

N = 4096
F = 512
H = 16
C = 16
B = 1024
I = N // B
P = I * (I + 1) // 2


def _pair(k):
    bi = jnp.int32(0)
    for t in range(1, I):
        bi = bi + (k >= t * I - t * (t - 1) // 2).astype(jnp.int32)
    bj = k - (bi * I - bi * (bi - 1) // 2) + bi
    return bi, bj


def _clip_block(ap_ref, masked):
    u = jnp.clip(ap_ref[...], 0.0, 1.0)
    if masked:
        r = jax.lax.broadcasted_iota(jnp.int32, (B, B), 0)
        c = jax.lax.broadcasted_iota(jnp.int32, (B, B), 1)
        u = jnp.where(r < c, u, 0.0)
    return u


def _deg_body(ap_ref, dis_ref, acc, accc):
    k = pl.program_id(0)
    bi, bj = _pair(k)

    @pl.when(k == 0)
    def _init():
        acc[...] = jnp.zeros_like(acc)
        accc[...] = jnp.zeros_like(accc)

    def contrib(masked):
        T = _clip_block(ap_ref, masked)
        acc[0, pl.ds(bj * B, B)] += jnp.sum(T, axis=0)
        accc[pl.ds(bi * B, B), :] += jnp.sum(T, axis=1, keepdims=True)

    @pl.when(bi == bj)
    def _diag():
        contrib(True)

    @pl.when(bi != bj)
    def _off():
        contrib(False)

    @pl.when(k == P - 1)
    def _fini():
        dis_ref[...] = jax.lax.rsqrt(acc[...] + accc[...].T + 1.0)


def _layer_body(first_mm, last_mm):
    def body(dis_ref, dense_refs, ap_ref, out_ref, u_s, uT_s, acc_s, accT_s):
        k = pl.program_id(0)
        bi, bj = _pair(k)

        @pl.when(k == 0)
        def _init():
            u = dis_ref[...] * first_mm(dense_refs)
            u_s[...] = u
            uT_s[...] = u.astype(jnp.bfloat16).T
            acc_s[...] = jnp.zeros_like(acc_s)
            accT_s[...] = jnp.zeros_like(accT_s)

        def contrib(masked):
            T = _clip_block(ap_ref, masked).astype(jnp.bfloat16)
            vj = u_s[pl.ds(bj * B, B), :].astype(jnp.bfloat16)
            viT = uT_s[:, pl.ds(bi * B, B)]
            acc_s[pl.ds(bi * B, B), :] += jnp.dot(
                T, vj, preferred_element_type=jnp.float32)
            accT_s[:, pl.ds(bj * B, B)] += jnp.dot(
                viT, T, preferred_element_type=jnp.float32)

        @pl.when(bi == bj)
        def _diag():
            contrib(True)

        @pl.when(bi != bj)
        def _off():
            contrib(False)

        @pl.when(k == P - 1)
        def _fini():
            tot = acc_s[...] + accT_s[...].T + u_s[...]
            out_ref[...] = last_mm(dense_refs, dis_ref[...] * tot)

    return body


def _full(shape):
    return pl.BlockSpec(shape, lambda k: (0,) * len(shape))


def _ap_spec():
    return pl.BlockSpec((B, B), _pair)


def kernel(x, A_param, W1, b1, W2, b2):
    assert x.shape == (N, F) and A_param.shape == (N, N)
    b1r = b1.reshape(1, H)
    b2r = b2.reshape(1, C)

    grid = (P,)

    dis_row = pl.pallas_call(
        _deg_body,
        grid=grid,
        in_specs=[_ap_spec()],
        out_specs=_full((1, N)),
        out_shape=jax.ShapeDtypeStruct((1, N), jnp.float32),
        scratch_shapes=[
            pltpu.VMEM((1, N), jnp.float32),
            pltpu.VMEM((N, 1), jnp.float32),
        ],
    )(A_param)
    dis = dis_row.reshape(N, 1)

    def l1_first(refs):
        x_ref, w1_ref, b1_ref, w2_ref = refs
        return jnp.dot(x_ref[...], w1_ref[...],
                       preferred_element_type=jnp.float32)

    def l1_last(refs, y):
        x_ref, w1_ref, b1_ref, w2_ref = refs
        h = jax.nn.relu(y + b1_ref[...])
        return jnp.dot(h, w2_ref[...], preferred_element_type=jnp.float32)

    def body1(dis_ref, x_ref, w1_ref, b1_ref, w2_ref, ap_ref, out_ref,
              u_s, uT_s, acc_s, accT_s):
        _layer_body(l1_first, l1_last)(
            dis_ref, (x_ref, w1_ref, b1_ref, w2_ref), ap_ref, out_ref,
            u_s, uT_s, acc_s, accT_s)

    v2 = pl.pallas_call(
        body1,
        grid=grid,
        in_specs=[_full((N, 1)), _full((N, F)), _full((F, H)),
                  _full((1, H)), _full((H, C)), _ap_spec()],
        out_specs=_full((N, C)),
        out_shape=jax.ShapeDtypeStruct((N, C), jnp.float32),
        scratch_shapes=[
            pltpu.VMEM((N, H), jnp.float32),
            pltpu.VMEM((H, N), jnp.bfloat16),
            pltpu.VMEM((N, H), jnp.float32),
            pltpu.VMEM((H, N), jnp.float32),
        ],
    )(dis, x, W1, b1r, W2, A_param)

    def l2_first(refs):
        (v2_ref, b2_ref) = refs
        return v2_ref[...]

    def l2_last(refs, y):
        (v2_ref, b2_ref) = refs
        return y + b2_ref[...]

    def body2(dis_ref, v2_ref, b2_ref, ap_ref, out_ref,
              u_s, uT_s, acc_s, accT_s):
        _layer_body(l2_first, l2_last)(
            dis_ref, (v2_ref, b2_ref), ap_ref, out_ref,
            u_s, uT_s, acc_s, accT_s)

    out = pl.pallas_call(
        body2,
        grid=grid,
        in_specs=[_full((N, 1)), _full((N, C)), _full((1, C)), _ap_spec()],
        out_specs=_full((N, C)),
        out_shape=jax.ShapeDtypeStruct((N, C), jnp.float32),
        scratch_shapes=[
            pltpu.VMEM((N, C), jnp.float32),
            pltpu.VMEM((C, N), jnp.bfloat16),
            pltpu.VMEM((N, C), jnp.float32),
            pltpu.VMEM((C, N), jnp.float32),
        ],
    )(dis, v2, b2r, A_param)

    return out
